# Initial kernel scaffold; baseline (speedup 1.0000x reference)
import jax
import jax.numpy as jnp
from jax.experimental import pallas as pl


def kernel(hidden, feat, proto_context, W_hctx, W_fctx, W_feat, b_feat, W_r1, b_r1, W_r2, b_r2, W_e1, b_e1, W_e2, b_e2):
    raise NotImplementedError("write your pallas kernel here")


# speedup vs baseline: 1.1130x; 1.1130x over previous
import jax
import jax.numpy as jnp
from jax.experimental import pallas as pl
from jax.experimental.pallas import tpu as pltpu

B, S = 2, 2048
T = B * S
D_MODEL = 1024
N_FEAT = 32
PROTO_DIM = 256
D_FEMB = 128
D_RH = 256
E = 8
DH = 1024
EPAD = 128

BLK_T = 256
NEG = -1e30


def _dense_body(hid, ft, proto, w_hctx, w_fctx, w_feat, b_feat,
                w_r1h, w_r1f, b_r1, w_r2p, b_r2p,
                w_e1, b_e1, w_e2, b_e2,
                sd, glp, gwp):
    proto_row = proto[0]
    hc = hid[...] + jnp.dot(proto_row, w_hctx[...],
                            preferred_element_type=jnp.float32)
    fc = ft[...] + jnp.dot(proto_row, w_fctx[...],
                           preferred_element_type=jnp.float32)
    fe = jax.nn.relu(jnp.dot(fc, w_feat[...],
                             preferred_element_type=jnp.float32) + b_feat[...])
    rh = jax.nn.relu(
        jnp.dot(hc, w_r1h[...], preferred_element_type=jnp.float32)
        + jnp.dot(fc, w_r1f[...], preferred_element_type=jnp.float32)
        + b_r1[...])
    lg = jnp.dot(rh, w_r2p[...], preferred_element_type=jnp.float32) + b_r2p[...]
    glp[...] = lg

    lanes = jax.lax.broadcasted_iota(jnp.int32, (BLK_T, EPAD), 1)
    v1 = jnp.max(lg, axis=1, keepdims=True)
    i1 = jnp.min(jnp.where(lg == v1, lanes, EPAD), axis=1, keepdims=True)
    lg2 = jnp.where(lanes == i1, NEG, lg)
    v2 = jnp.max(lg2, axis=1, keepdims=True)
    i2 = jnp.min(jnp.where(lg2 == v2, lanes, EPAD), axis=1, keepdims=True)
    w1 = 1.0 / (1.0 + jnp.exp(v2 - v1))
    w2 = 1.0 - w1
    gw = jnp.where(lanes == i1, w1, 0.0) + jnp.where(lanes == i2, w2, 0.0)
    gwp[...] = gw

    hcb = hc.astype(jnp.bfloat16)
    feb = fe.astype(jnp.bfloat16)
    acc = jnp.zeros((BLK_T, D_MODEL), jnp.float32)
    for e in range(E):
        h1 = jax.nn.relu(
            jnp.dot(hcb, w_e1[e, :D_MODEL, :], preferred_element_type=jnp.float32)
            + jnp.dot(feb, w_e1[e, D_MODEL:, :], preferred_element_type=jnp.float32)
            + b_e1[e])
        oe = jnp.dot(h1.astype(jnp.bfloat16), w_e2[e],
                     preferred_element_type=jnp.float32) + b_e2[e]
        acc = acc + gw[:, e:e + 1] * oe
    sd[...] = acc


def kernel(hidden, feat, proto_context, W_hctx, W_fctx, W_feat, b_feat,
           W_r1, b_r1, W_r2, b_r2, W_e1, b_e1, W_e2, b_e2):
    hid = hidden.reshape(T, D_MODEL)
    ft = feat.reshape(T, N_FEAT)
    proto = proto_context.reshape(B, 1, PROTO_DIM)
    w_r1h = W_r1[:D_MODEL]
    w_r1f = W_r1[D_MODEL:]
    w_r2p = jnp.zeros((D_RH, EPAD), jnp.float32).at[:, :E].set(W_r2)
    b_r2p = jnp.full((1, EPAD), NEG, jnp.float32).at[0, :E].set(b_r2)
    w_e1 = W_e1.astype(jnp.bfloat16)
    w_e2 = W_e2.astype(jnp.bfloat16)
    b_e1 = b_e1.reshape(E, 1, DH)
    b_e2 = b_e2.reshape(E, 1, D_MODEL)

    nblk = T // BLK_T
    const = lambda *shp: pl.BlockSpec(shp, lambda i: (0,) * len(shp))
    grid_spec = pl.GridSpec(
        grid=(nblk,),
        in_specs=[
            pl.BlockSpec((BLK_T, D_MODEL), lambda i: (i, 0)),
            pl.BlockSpec((BLK_T, N_FEAT), lambda i: (i, 0)),
            pl.BlockSpec((1, 1, PROTO_DIM), lambda i: (i // (S // BLK_T), 0, 0)),
            const(PROTO_DIM, D_MODEL),
            const(PROTO_DIM, N_FEAT),
            const(N_FEAT, D_FEMB),
            const(1, D_FEMB),
            const(D_MODEL, D_RH),
            const(N_FEAT, D_RH),
            const(1, D_RH),
            const(D_RH, EPAD),
            const(1, EPAD),
            const(E, D_MODEL + D_FEMB, DH),
            const(E, 1, DH),
            const(E, DH, D_MODEL),
            const(E, 1, D_MODEL),
        ],
        out_specs=[
            pl.BlockSpec((BLK_T, D_MODEL), lambda i: (i, 0)),
            pl.BlockSpec((BLK_T, EPAD), lambda i: (i, 0)),
            pl.BlockSpec((BLK_T, EPAD), lambda i: (i, 0)),
        ],
    )
    sd, glp, gwp = pl.pallas_call(
        _dense_body,
        grid_spec=grid_spec,
        out_shape=[
            jax.ShapeDtypeStruct((T, D_MODEL), jnp.float32),
            jax.ShapeDtypeStruct((T, EPAD), jnp.float32),
            jax.ShapeDtypeStruct((T, EPAD), jnp.float32),
        ],
    )(hid, ft, proto, W_hctx, W_fctx, W_feat, b_feat.reshape(1, D_FEMB),
      w_r1h, w_r1f, b_r1.reshape(1, D_RH), w_r2p, b_r2p,
      w_e1, b_e1, w_e2, b_e2)

    stage_delta = sd.reshape(B, S, D_MODEL)
    gate_logits = glp[:, :E].reshape(B, S, E)
    gate_weights = gwp[:, :E].reshape(B, S, E)
    return stage_delta, gate_weights, gate_logits
